# baseline (speedup 1.0000x reference)
import functools
import numpy as np
import jax
import jax.numpy as jnp
from jax import lax
from jax.experimental import pallas as pl
from jax.experimental.pallas import tpu as pltpu
from jax.experimental.pallas import tpu_sc as plsc

DIM = 128
NDIM = 3
NUM_GRID = 32 * 32 * 32
N_MESH = 100000
N_MESH_PAD = 100352
N_EDGES = 600000

NC, NS = 2, 16
NW = NC * NS
CH = 128
CHUNKS = 147
EPW = CH * CHUNKS
E_PAD = NW * EPW


def _embed_consts():
  eff = 42
  omega = 1.0 / (10000.0 ** (np.arange(0, eff, 2, dtype=np.float32) / eff))
  A = np.zeros((NDIM, DIM), np.float32)
  PH = np.zeros((DIM,), np.float32)
  for p in range(126):
    d, r = p // 42, p % 42
    if r < 21:
      A[d, p] = omega[r]
    else:
      A[d, p] = omega[r - 21]
      PH[p] = np.pi / 2
  return jnp.asarray(A), jnp.asarray(PH).reshape(1, DIM)


def _embed_proj(pos, W, b, blk=512):
  N = pos.shape[0]
  A, PH = _embed_consts()
  Dout = W.shape[1]

  def body(pos_ref, A_ref, PH_ref, W_ref, b_ref, o_ref):
    emb = jnp.sin(
        lax.dot(pos_ref[...], A_ref[...],
                preferred_element_type=jnp.float32) + PH_ref[...])
    o_ref[...] = lax.dot(
        emb, W_ref[...], preferred_element_type=jnp.float32) + b_ref[...]

  return pl.pallas_call(
      body,
      grid=(N // blk,),
      in_specs=[
          pl.BlockSpec((blk, NDIM), lambda i: (i, 0)),
          pl.BlockSpec((NDIM, DIM), lambda i: (0, 0)),
          pl.BlockSpec((1, DIM), lambda i: (0, 0)),
          pl.BlockSpec((DIM, Dout), lambda i: (0, 0)),
          pl.BlockSpec((1, Dout), lambda i: (0, 0)),
      ],
      out_specs=pl.BlockSpec((blk, Dout), lambda i: (i, 0)),
      out_shape=jax.ShapeDtypeStruct((N, Dout), jnp.float32),
  )(pos, A, PH, W, b.reshape(1, Dout))


def _sc_gather(h_mesh, h_grid, midx, gidx):
  D = h_mesh.shape[1]
  mesh = plsc.VectorSubcoreMesh(
      core_axis_name="c", subcore_axis_name="s", num_cores=NC, num_subcores=NS)

  @functools.partial(
      pl.kernel,
      out_type=[
          jax.ShapeDtypeStruct((E_PAD, D), jnp.float32),
          jax.ShapeDtypeStruct((E_PAD, D), jnp.float32),
      ],
      mesh=mesh,
      scratch_types=[
          pltpu.VMEM((CHUNKS, CH), jnp.int32),
          pltpu.VMEM((CHUNKS, CH), jnp.int32),
          pltpu.VMEM((CH, D), jnp.float32),
          pltpu.VMEM((CH, D), jnp.float32),
          pltpu.SemaphoreType.DMA,
          pltpu.SemaphoreType.DMA,
      ],
  )
  def k(hm, hg, mi, gi, out_m, out_g, mi_v, gi_v, mrow, grow, sem1, sem2):
    wid = lax.axis_index("s") * NC + lax.axis_index("c")
    base = wid * EPW
    pltpu.sync_copy(mi.at[wid], mi_v)
    pltpu.sync_copy(gi.at[wid], gi_v)

    def body(kk, carry):
      mcp = pltpu.async_copy(hm.at[mi_v.at[kk]], mrow, sem1)
      gcp = pltpu.async_copy(hg.at[gi_v.at[kk]], grow, sem2)
      mcp.wait()
      gcp.wait()
      pltpu.sync_copy(mrow, out_m.at[pl.ds(base + kk * CH, CH)])
      pltpu.sync_copy(grow, out_g.at[pl.ds(base + kk * CH, CH)])
      return carry

    lax.fori_loop(0, CHUNKS, body, 0)

  return k(h_mesh, h_grid, midx.reshape(NW, CHUNKS, CH),
           gidx.reshape(NW, CHUNKS, CH))


CC = 128
NCHUNK = NUM_GRID // CC
CPW = NCHUNK // NW
ECH = 128


def _sc_segsum(g2v, gidx, indptr):
  mesh = plsc.VectorSubcoreMesh(
      core_axis_name="c", subcore_axis_name="s", num_cores=NC, num_subcores=NS)

  @functools.partial(
      pl.kernel,
      out_type=[
          jax.ShapeDtypeStruct((NUM_GRID * 2, 128), jnp.float32),
          jax.ShapeDtypeStruct((NCHUNK * 8, 16), jnp.float32),
      ],
      mesh=mesh,
      scratch_types=[
          pltpu.VMEM((CC + 16,), jnp.int32),
          pltpu.VMEM((ECH + 16,), jnp.int32),
          pltpu.VMEM((CC * 2, 128), jnp.float32),
          pltpu.VMEM((ECH * 2, 128), jnp.float32),
          pltpu.VMEM((CC // 16, 16), jnp.float32),
      ],
  )
  def k(g2, gi, ip, sums, counts, ip_v, gi_s, accum, ebuf, cacc):
    wid = lax.axis_index("s") * NC + lax.axis_index("c")
    zero16 = jnp.zeros((16,), jnp.float32)

    for j in range(CPW):
      c = wid * CPW + j
      c0 = c * CC
      pltpu.sync_copy(ip.at[pl.ds(pl.multiple_of(c0, 8), CC + 8)],
                      ip_v.at[pl.ds(0, CC + 8)])
      estart = ip_v[pl.ds(0, 16)][0]
      eend = ip_v[pl.ds(CC, 16)][0]

      def zbody(i, carry):
        for q in range(8):
          accum[i, pl.ds(q * 16, 16)] = zero16
        return carry
      lax.fori_loop(0, CC * 2, zbody, 0)

      for gph in range(CC // 16):
        hi_w = ip_v[pl.ds(gph * 16 + 1, 16)]
        lo_w = ip_v[pl.ds(gph * 16, 16)]
        cacc[gph] = (hi_w - lo_w).astype(jnp.float32)

      abase = pl.multiple_of((estart // 8) * 8, 8)

      def sub_chunk(s, carry):
        ebase = abase + s * ECH
        pltpu.sync_copy(
            g2.at[pl.ds(pl.multiple_of(ebase * 2, 16), ECH * 2)], ebuf)
        pltpu.sync_copy(gi.at[pl.ds(pl.multiple_of(ebase, 8), ECH)],
                        gi_s.at[pl.ds(0, ECH)])
        lo = jnp.maximum(estart - ebase, 0)
        hi = jnp.minimum(eend - ebase, ECH)

        @pl.when(hi > lo)
        def _():
          cc_lo = gi_s[pl.ds(lo, 16)][0] - c0
          cc_hi = gi_s[pl.ds(hi - 1, 16)][0] - c0

          def cell(cc, ccarry):
            e_s = jnp.maximum(ip_v[pl.ds(cc, 16)][0] - ebase, lo)
            e_e = jnp.minimum(ip_v[pl.ds(cc + 1, 16)][0] - ebase, hi)

            def eb(i, acc):
              return tuple(
                  acc[g] + ebuf[i * 2 + g // 8, pl.ds((g % 8) * 16, 16)]
                  for g in range(16))

            acc = lax.fori_loop(e_s, e_e, eb, (zero16,) * 16)
            for g in range(16):
              sl = pl.ds((g % 8) * 16, 16)
              row = cc * 2 + g // 8
              accum[row, sl] = accum[row, sl] + acc[g]
            return ccarry

          lax.fori_loop(cc_lo, cc_hi + 1, cell, 0)
        return carry

      nsub = (eend - abase + ECH - 1) // ECH
      lax.fori_loop(0, nsub, sub_chunk, 0)

      pltpu.sync_copy(
          accum, sums.at[pl.ds(pl.multiple_of(c0 * 2, 256), CC * 2)])
      pltpu.sync_copy(
          cacc, counts.at[pl.ds(pl.multiple_of(c * 8, 8), CC // 16)])

  return k(g2v, gidx, indptr)


def _edge_mlp(em, eg, W2, b2, blk=512):
  E, D = em.shape

  def body(em_ref, eg_ref, W_ref, b_ref, o_ref):
    x = em_ref[...] + eg_ref[...]
    x = 0.5 * x * (1.0 + lax.erf(x * (2.0 ** -0.5)))
    y = lax.dot(x.astype(jnp.bfloat16), W_ref[...].astype(jnp.bfloat16),
                preferred_element_type=jnp.float32) + b_ref[...]
    o_ref[...] = 0.5 * y * (1.0 + lax.erf(y * (2.0 ** -0.5)))

  return pl.pallas_call(
      body,
      grid=(E // blk,),
      in_specs=[
          pl.BlockSpec((blk, D), lambda i: (i, 0)),
          pl.BlockSpec((blk, D), lambda i: (i, 0)),
          pl.BlockSpec((D, D), lambda i: (0, 0)),
          pl.BlockSpec((1, D), lambda i: (0, 0)),
      ],
      out_specs=pl.BlockSpec((blk, D), lambda i: (i, 0)),
      out_shape=jax.ShapeDtypeStruct((E, D), jnp.float32),
  )(em, eg, W2, b2.reshape(1, D))


def _mean_proj(sums, counts, W3, b3, blk=512):
  G, D = sums.shape
  Dout = W3.shape[1]

  def body(s_ref, c_ref, W_ref, b_ref, o_ref):
    c = c_ref[...]
    mean = s_ref[...] / jnp.maximum(c, 1.0)
    o_ref[...] = (lax.dot(mean, W_ref[...], preferred_element_type=jnp.float32)
                  + b_ref[...] * (c > 0))

  return pl.pallas_call(
      body,
      grid=(G // blk,),
      in_specs=[
          pl.BlockSpec((blk, D), lambda i: (i, 0)),
          pl.BlockSpec((blk, 1), lambda i: (i, 0)),
          pl.BlockSpec((D, Dout), lambda i: (0, 0)),
          pl.BlockSpec((1, Dout), lambda i: (0, 0)),
      ],
      out_specs=pl.BlockSpec((blk, Dout), lambda i: (i, 0)),
      out_shape=jax.ShapeDtypeStruct((G, Dout), jnp.float32),
  )(sums, counts.reshape(G, 1), W3, b3.reshape(1, Dout))


def kernel(mesh_pos, grid_pos, mesh_to_grid_edges, W1, b1, W2, b2, W3, b3):
  grid_idx = mesh_to_grid_edges[:, 0].astype(jnp.int32)
  mesh_idx = mesh_to_grid_edges[:, 1].astype(jnp.int32)

  mp = jnp.concatenate(
      [mesh_pos, jnp.zeros((N_MESH_PAD - N_MESH, NDIM), jnp.float32)])
  h_mesh = _embed_proj(mp, W1[:DIM], jnp.zeros_like(b1))[:N_MESH]
  h_grid = _embed_proj(grid_pos, W1[DIM:], b1)

  midx_p = jnp.concatenate(
      [mesh_idx, jnp.zeros((E_PAD - N_EDGES,), jnp.int32)])
  gidx_p = jnp.concatenate(
      [grid_idx, jnp.full((E_PAD - N_EDGES,), NUM_GRID - 1, jnp.int32)])

  edge_m, edge_g = _sc_gather(h_mesh, h_grid, midx_p, gidx_p)

  g2 = _edge_mlp(edge_m, edge_g, W2, b2)

  indptr = jnp.searchsorted(
      grid_idx, jnp.arange(0, NUM_GRID + 8, dtype=jnp.int32),
      side="left").astype(jnp.int32)

  sums_v, counts_v = _sc_segsum(
      g2.reshape(E_PAD * 2, 128), gidx_p, indptr)
  sums = sums_v.reshape(NUM_GRID, DIM * 2)
  counts = counts_v.reshape(NUM_GRID)

  out = _mean_proj(sums, counts, W3, b3)
  return out.reshape(-1, NUM_GRID, DIM)


# speedup vs baseline: 1.4012x; 1.4012x over previous
import functools
import numpy as np
import jax
import jax.numpy as jnp
from jax import lax
from jax.experimental import pallas as pl
from jax.experimental.pallas import tpu as pltpu
from jax.experimental.pallas import tpu_sc as plsc

DIM = 128
NDIM = 3
NUM_GRID = 32 * 32 * 32
N_MESH = 100000
N_MESH_PAD = 100352
N_EDGES = 600000

NC, NS = 2, 16
NW = NC * NS
CH = 128
CHUNKS = 147
EPW = CH * CHUNKS
E_PAD = NW * EPW


def _embed_consts():
  eff = 42
  omega = 1.0 / (10000.0 ** (np.arange(0, eff, 2, dtype=np.float32) / eff))
  A = np.zeros((NDIM, DIM), np.float32)
  PH = np.zeros((DIM,), np.float32)
  for p in range(126):
    d, r = p // 42, p % 42
    if r < 21:
      A[d, p] = omega[r]
    else:
      A[d, p] = omega[r - 21]
      PH[p] = np.pi / 2
  return jnp.asarray(A), jnp.asarray(PH).reshape(1, DIM)


def _embed_proj(pos, W, b, blk=512):
  N = pos.shape[0]
  A, PH = _embed_consts()
  Dout = W.shape[1]

  def body(pos_ref, A_ref, PH_ref, W_ref, b_ref, o_ref):
    emb = jnp.sin(
        lax.dot(pos_ref[...], A_ref[...],
                preferred_element_type=jnp.float32) + PH_ref[...])
    o_ref[...] = lax.dot(
        emb, W_ref[...], preferred_element_type=jnp.float32) + b_ref[...]

  return pl.pallas_call(
      body,
      grid=(N // blk,),
      in_specs=[
          pl.BlockSpec((blk, NDIM), lambda i: (i, 0)),
          pl.BlockSpec((NDIM, DIM), lambda i: (0, 0)),
          pl.BlockSpec((1, DIM), lambda i: (0, 0)),
          pl.BlockSpec((DIM, Dout), lambda i: (0, 0)),
          pl.BlockSpec((1, Dout), lambda i: (0, 0)),
      ],
      out_specs=pl.BlockSpec((blk, Dout), lambda i: (i, 0)),
      out_shape=jax.ShapeDtypeStruct((N, Dout), jnp.float32),
  )(pos, A, PH, W, b.reshape(1, Dout))


def _sc_gather(h_mesh, h_grid, midx, gidx):
  D = h_mesh.shape[1]
  mesh = plsc.VectorSubcoreMesh(
      core_axis_name="c", subcore_axis_name="s", num_cores=NC, num_subcores=NS)

  @functools.partial(
      pl.kernel,
      out_type=[
          jax.ShapeDtypeStruct((E_PAD, D), jnp.float32),
          jax.ShapeDtypeStruct((E_PAD, D), jnp.float32),
      ],
      mesh=mesh,
      scratch_types=[
          pltpu.VMEM((CHUNKS, CH), jnp.int32),
          pltpu.VMEM((CHUNKS, CH), jnp.int32),
          pltpu.VMEM((CH, D), jnp.float32),
          pltpu.VMEM((CH, D), jnp.float32),
          pltpu.SemaphoreType.DMA,
          pltpu.SemaphoreType.DMA,
      ],
  )
  def k(hm, hg, mi, gi, out_m, out_g, mi_v, gi_v, mrow, grow, sem1, sem2):
    wid = lax.axis_index("s") * NC + lax.axis_index("c")
    base = wid * EPW
    pltpu.sync_copy(mi.at[wid], mi_v)
    pltpu.sync_copy(gi.at[wid], gi_v)

    def body(kk, carry):
      mcp = pltpu.async_copy(hm.at[mi_v.at[kk]], mrow, sem1)
      gcp = pltpu.async_copy(hg.at[gi_v.at[kk]], grow, sem2)
      mcp.wait()
      gcp.wait()
      pltpu.sync_copy(mrow, out_m.at[pl.ds(base + kk * CH, CH)])
      pltpu.sync_copy(grow, out_g.at[pl.ds(base + kk * CH, CH)])
      return carry

    lax.fori_loop(0, CHUNKS, body, 0)

  return k(h_mesh, h_grid, midx.reshape(NW, CHUNKS, CH),
           gidx.reshape(NW, CHUNKS, CH))


CC = 128
NCHUNK = NUM_GRID // CC
CPW = NCHUNK // NW
ECH = 128


def _sc_segsum(g2v, gidx, indptr):
  mesh = plsc.VectorSubcoreMesh(
      core_axis_name="c", subcore_axis_name="s", num_cores=NC, num_subcores=NS)

  @functools.partial(
      pl.kernel,
      out_type=[
          jax.ShapeDtypeStruct((NUM_GRID * 2, 128), jnp.float32),
          jax.ShapeDtypeStruct((NCHUNK * 8, 16), jnp.float32),
      ],
      mesh=mesh,
      scratch_types=[
          pltpu.VMEM((CC + 16,), jnp.int32),
          pltpu.VMEM((ECH + 16,), jnp.int32),
          pltpu.VMEM((CC * 2, 128), jnp.float32),
          pltpu.VMEM((ECH * 2, 128), jnp.float32),
          pltpu.VMEM((CC // 16, 16), jnp.float32),
      ],
  )
  def k(g2, gi, ip, sums, counts, ip_v, gi_s, accum, ebuf, cacc):
    wid = lax.axis_index("s") * NC + lax.axis_index("c")
    zero16 = jnp.zeros((16,), jnp.float32)

    for j in range(CPW):
      c = wid * CPW + j
      c0 = c * CC
      pltpu.sync_copy(ip.at[pl.ds(pl.multiple_of(c0, 8), CC + 8)],
                      ip_v.at[pl.ds(0, CC + 8)])
      estart = ip_v[pl.ds(0, 16)][0]
      eend = ip_v[pl.ds(CC, 16)][0]

      def zbody(i, carry):
        for q in range(8):
          accum[i, pl.ds(q * 16, 16)] = zero16
        return carry
      lax.fori_loop(0, CC * 2, zbody, 0)

      for gph in range(CC // 16):
        hi_w = ip_v[pl.ds(gph * 16 + 1, 16)]
        lo_w = ip_v[pl.ds(gph * 16, 16)]
        cacc[gph] = (hi_w - lo_w).astype(jnp.float32)

      abase = pl.multiple_of((estart // 8) * 8, 8)

      def sub_chunk(s, carry):
        ebase = abase + s * ECH
        pltpu.sync_copy(
            g2.at[pl.ds(pl.multiple_of(ebase * 2, 16), ECH * 2)], ebuf)
        pltpu.sync_copy(gi.at[pl.ds(pl.multiple_of(ebase, 8), ECH)],
                        gi_s.at[pl.ds(0, ECH)])
        lo = jnp.maximum(estart - ebase, 0)
        hi = jnp.minimum(eend - ebase, ECH)

        @pl.when(hi > lo)
        def _():
          cc_lo = gi_s[pl.ds(lo, 16)][0] - c0
          cc_hi = gi_s[pl.ds(hi - 1, 16)][0] - c0

          def cell(cc, ccarry):
            e_s = jnp.maximum(ip_v[pl.ds(cc, 16)][0] - ebase, lo)
            e_e = jnp.minimum(ip_v[pl.ds(cc + 1, 16)][0] - ebase, hi)

            def eb(i, acc):
              return tuple(
                  acc[g] + ebuf[i * 2 + g // 8, pl.ds((g % 8) * 16, 16)]
                  for g in range(16))

            acc = lax.fori_loop(e_s, e_e, eb, (zero16,) * 16)
            for g in range(16):
              sl = pl.ds((g % 8) * 16, 16)
              row = cc * 2 + g // 8
              accum[row, sl] = accum[row, sl] + acc[g]
            return ccarry

          lax.fori_loop(cc_lo, cc_hi + 1, cell, 0)
        return carry

      nsub = (eend - abase + ECH - 1) // ECH
      lax.fori_loop(0, nsub, sub_chunk, 0)

      pltpu.sync_copy(
          accum, sums.at[pl.ds(pl.multiple_of(c0 * 2, 256), CC * 2)])
      pltpu.sync_copy(
          cacc, counts.at[pl.ds(pl.multiple_of(c * 8, 8), CC // 16)])

  return k(g2v, gidx, indptr)


def _edge_mlp(em, eg, W2, b2, blk=512):
  E, D = em.shape

  def body(em_ref, eg_ref, W_ref, b_ref, o_ref):
    x = em_ref[...] + eg_ref[...]
    x = 0.5 * x * (1.0 + lax.erf(x * (2.0 ** -0.5)))
    y = lax.dot(x.astype(jnp.bfloat16), W_ref[...].astype(jnp.bfloat16),
                preferred_element_type=jnp.float32) + b_ref[...]
    o_ref[...] = 0.5 * y * (1.0 + lax.erf(y * (2.0 ** -0.5)))

  return pl.pallas_call(
      body,
      grid=(E // blk,),
      in_specs=[
          pl.BlockSpec((blk, D), lambda i: (i, 0)),
          pl.BlockSpec((blk, D), lambda i: (i, 0)),
          pl.BlockSpec((D, D), lambda i: (0, 0)),
          pl.BlockSpec((1, D), lambda i: (0, 0)),
      ],
      out_specs=pl.BlockSpec((blk, D), lambda i: (i, 0)),
      out_shape=jax.ShapeDtypeStruct((E, D), jnp.float32),
  )(em, eg, W2, b2.reshape(1, D))


def _mean_proj(sums, counts, W3, b3, blk=512):
  G, D = sums.shape
  Dout = W3.shape[1]

  def body(s_ref, c_ref, W_ref, b_ref, o_ref):
    c = c_ref[...]
    mean = s_ref[...] / jnp.maximum(c, 1.0)
    o_ref[...] = (lax.dot(mean, W_ref[...], preferred_element_type=jnp.float32)
                  + b_ref[...] * (c > 0))

  return pl.pallas_call(
      body,
      grid=(G // blk,),
      in_specs=[
          pl.BlockSpec((blk, D), lambda i: (i, 0)),
          pl.BlockSpec((blk, 1), lambda i: (i, 0)),
          pl.BlockSpec((D, Dout), lambda i: (0, 0)),
          pl.BlockSpec((1, Dout), lambda i: (0, 0)),
      ],
      out_specs=pl.BlockSpec((blk, Dout), lambda i: (i, 0)),
      out_shape=jax.ShapeDtypeStruct((G, Dout), jnp.float32),
  )(sums, counts.reshape(G, 1), W3, b3.reshape(1, Dout))


def kernel(mesh_pos, grid_pos, mesh_to_grid_edges, W1, b1, W2, b2, W3, b3):
  grid_idx = mesh_to_grid_edges[:, 0].astype(jnp.int32)
  mesh_idx = mesh_to_grid_edges[:, 1].astype(jnp.int32)

  mp = jnp.concatenate(
      [mesh_pos, jnp.zeros((N_MESH_PAD - N_MESH, NDIM), jnp.float32)])
  h_mesh = _embed_proj(mp, W1[:DIM], jnp.zeros_like(b1))[:N_MESH]
  h_grid = _embed_proj(grid_pos, W1[DIM:], b1)

  midx_p = jnp.concatenate(
      [mesh_idx, jnp.zeros((E_PAD - N_EDGES,), jnp.int32)])
  gidx_p = jnp.concatenate(
      [grid_idx, jnp.full((E_PAD - N_EDGES,), NUM_GRID - 1, jnp.int32)])

  edge_m, edge_g = _sc_gather(h_mesh, h_grid, midx_p, gidx_p)

  g2 = _edge_mlp(edge_m, edge_g, W2, b2)

  cnt = jnp.zeros((NUM_GRID,), jnp.int32).at[grid_idx].add(
      1, mode="drop", unique_indices=False, indices_are_sorted=True)
  indptr = jnp.concatenate([
      jnp.zeros((1,), jnp.int32),
      jnp.cumsum(cnt, dtype=jnp.int32),
      jnp.full((7,), N_EDGES, jnp.int32),
  ])

  sums_v, counts_v = _sc_segsum(
      g2.reshape(E_PAD * 2, 128), gidx_p, indptr)
  sums = sums_v.reshape(NUM_GRID, DIM * 2)
  counts = counts_v.reshape(NUM_GRID)

  out = _mean_proj(sums, counts, W3, b3)
  return out.reshape(-1, NUM_GRID, DIM)


# speedup vs baseline: 1.4877x; 1.0617x over previous
import functools
import numpy as np
import jax
import jax.numpy as jnp
from jax import lax
from jax.experimental import pallas as pl
from jax.experimental.pallas import tpu as pltpu
from jax.experimental.pallas import tpu_sc as plsc

DIM = 128
NDIM = 3
NUM_GRID = 32 * 32 * 32
N_MESH = 100000
N_MESH_PAD = 100352
N_EDGES = 600000

NC, NS = 2, 16
NW = NC * NS
GCH = 96
GCHUNKS = 196
EPW = GCH * GCHUNKS
E_PAD = NW * EPW


def _embed_consts():
  eff = 42
  omega = 1.0 / (10000.0 ** (np.arange(0, eff, 2, dtype=np.float32) / eff))
  A = np.zeros((NDIM, DIM), np.float32)
  PH = np.zeros((DIM,), np.float32)
  for p in range(126):
    d, r = p // 42, p % 42
    if r < 21:
      A[d, p] = omega[r]
    else:
      A[d, p] = omega[r - 21]
      PH[p] = np.pi / 2
  return jnp.asarray(A), jnp.asarray(PH).reshape(1, DIM)


def _embed_proj(pos, W, b, blk=512):
  N = pos.shape[0]
  A, PH = _embed_consts()
  Dout = W.shape[1]

  def body(pos_ref, A_ref, PH_ref, W_ref, b_ref, o_ref):
    emb = jnp.sin(
        lax.dot(pos_ref[...], A_ref[...],
                preferred_element_type=jnp.float32) + PH_ref[...])
    o_ref[...] = lax.dot(
        emb, W_ref[...], preferred_element_type=jnp.float32) + b_ref[...]

  return pl.pallas_call(
      body,
      grid=(N // blk,),
      in_specs=[
          pl.BlockSpec((blk, NDIM), lambda i: (i, 0)),
          pl.BlockSpec((NDIM, DIM), lambda i: (0, 0)),
          pl.BlockSpec((1, DIM), lambda i: (0, 0)),
          pl.BlockSpec((DIM, Dout), lambda i: (0, 0)),
          pl.BlockSpec((1, Dout), lambda i: (0, 0)),
      ],
      out_specs=pl.BlockSpec((blk, Dout), lambda i: (i, 0)),
      out_shape=jax.ShapeDtypeStruct((N, Dout), jnp.float32),
  )(pos, A, PH, W, b.reshape(1, Dout))


def _sc_gather(h_mesh, h_grid, midx, gidx):
  D = h_mesh.shape[1]
  mesh = plsc.VectorSubcoreMesh(
      core_axis_name="c", subcore_axis_name="s", num_cores=NC, num_subcores=NS)

  @functools.partial(
      pl.kernel,
      out_type=jax.ShapeDtypeStruct((E_PAD, D), jnp.float32),
      mesh=mesh,
      scratch_types=[
          pltpu.VMEM((GCHUNKS, GCH), jnp.int32),
          pltpu.VMEM((GCHUNKS, GCH), jnp.int32),
          pltpu.VMEM((GCH, D), jnp.float32),
          pltpu.VMEM((GCH, D), jnp.float32),
          pltpu.VMEM((GCH, D), jnp.float32),
          pltpu.SemaphoreType.DMA,
          pltpu.SemaphoreType.DMA,
          pltpu.SemaphoreType.DMA,
      ],
  )
  def k(hm, hg, mi, gi, out, mi_v, gi_v, mrow, grow, obuf, sem1, sem2, osem):
    wid = lax.axis_index("s") * NC + lax.axis_index("c")
    base = wid * EPW
    pltpu.sync_copy(mi.at[wid], mi_v)
    pltpu.sync_copy(gi.at[wid], gi_v)

    def body(kk, carry):
      mcp = pltpu.async_copy(hm.at[mi_v.at[kk]], mrow, sem1)
      gcp = pltpu.async_copy(hg.at[gi_v.at[kk]], grow, sem2)

      @pl.when(kk > 0)
      def _():
        pltpu.make_async_copy(out.at[pl.ds(0, GCH)], obuf, osem).wait()

      mcp.wait()
      gcp.wait()

      def add_row(r, c2):
        for q in range(D // 16):
          sl = pl.ds(q * 16, 16)
          obuf[r, sl] = mrow[r, sl] + grow[r, sl]
        return c2
      lax.fori_loop(0, GCH, add_row, 0)

      pltpu.make_async_copy(
          obuf, out.at[pl.ds(base + kk * GCH, GCH)], osem).start()
      return carry

    lax.fori_loop(0, GCHUNKS, body, 0)
    pltpu.make_async_copy(out.at[pl.ds(0, GCH)], obuf, osem).wait()

  return k(h_mesh, h_grid, midx.reshape(NW, GCHUNKS, GCH),
           gidx.reshape(NW, GCHUNKS, GCH))


CC = 128
NCHUNK = NUM_GRID // CC
CPW = NCHUNK // NW
ECH = 128


def _sc_segsum(g2v, gidx, indptr):
  mesh = plsc.VectorSubcoreMesh(
      core_axis_name="c", subcore_axis_name="s", num_cores=NC, num_subcores=NS)

  @functools.partial(
      pl.kernel,
      out_type=[
          jax.ShapeDtypeStruct((NUM_GRID * 2, 128), jnp.float32),
          jax.ShapeDtypeStruct((NCHUNK * 8, 16), jnp.float32),
      ],
      mesh=mesh,
      scratch_types=[
          pltpu.VMEM((CC + 16,), jnp.int32),
          pltpu.VMEM((ECH + 16,), jnp.int32),
          pltpu.VMEM((CC * 2, 128), jnp.float32),
          pltpu.VMEM((ECH * 2, 128), jnp.float32),
          pltpu.VMEM((CC // 16, 16), jnp.float32),
      ],
  )
  def k(g2, gi, ip, sums, counts, ip_v, gi_s, accum, ebuf, cacc):
    wid = lax.axis_index("s") * NC + lax.axis_index("c")
    zero16 = jnp.zeros((16,), jnp.float32)

    for j in range(CPW):
      c = wid * CPW + j
      c0 = c * CC
      pltpu.sync_copy(ip.at[pl.ds(pl.multiple_of(c0, 8), CC + 8)],
                      ip_v.at[pl.ds(0, CC + 8)])
      estart = ip_v[pl.ds(0, 16)][0]
      eend = ip_v[pl.ds(CC, 16)][0]

      def zbody(i, carry):
        for q in range(8):
          accum[i, pl.ds(q * 16, 16)] = zero16
        return carry
      lax.fori_loop(0, CC * 2, zbody, 0)

      for gph in range(CC // 16):
        hi_w = ip_v[pl.ds(gph * 16 + 1, 16)]
        lo_w = ip_v[pl.ds(gph * 16, 16)]
        cacc[gph] = (hi_w - lo_w).astype(jnp.float32)

      abase = pl.multiple_of((estart // 8) * 8, 8)

      def sub_chunk(s, carry):
        ebase = abase + s * ECH
        pltpu.sync_copy(
            g2.at[pl.ds(pl.multiple_of(ebase * 2, 16), ECH * 2)], ebuf)
        pltpu.sync_copy(gi.at[pl.ds(pl.multiple_of(ebase, 8), ECH)],
                        gi_s.at[pl.ds(0, ECH)])
        lo = jnp.maximum(estart - ebase, 0)
        hi = jnp.minimum(eend - ebase, ECH)

        @pl.when(hi > lo)
        def _():
          cc_lo = gi_s[pl.ds(lo, 16)][0] - c0
          cc_hi = gi_s[pl.ds(hi - 1, 16)][0] - c0

          def cell(cc, ccarry):
            e_s = jnp.maximum(ip_v[pl.ds(cc, 16)][0] - ebase, lo)
            e_e = jnp.minimum(ip_v[pl.ds(cc + 1, 16)][0] - ebase, hi)

            def eb(i, acc):
              return tuple(
                  acc[g] + ebuf[i * 2 + g // 8, pl.ds((g % 8) * 16, 16)]
                  for g in range(16))

            acc = lax.fori_loop(e_s, e_e, eb, (zero16,) * 16)
            for g in range(16):
              sl = pl.ds((g % 8) * 16, 16)
              row = cc * 2 + g // 8
              accum[row, sl] = accum[row, sl] + acc[g]
            return ccarry

          lax.fori_loop(cc_lo, cc_hi + 1, cell, 0)
        return carry

      nsub = (eend - abase + ECH - 1) // ECH
      lax.fori_loop(0, nsub, sub_chunk, 0)

      pltpu.sync_copy(
          accum, sums.at[pl.ds(pl.multiple_of(c0 * 2, 256), CC * 2)])
      pltpu.sync_copy(
          cacc, counts.at[pl.ds(pl.multiple_of(c * 8, 8), CC // 16)])

  return k(g2v, gidx, indptr)


def _edge_mlp(ein, W2, b2, blk=512):
  E, D = ein.shape

  def body(e_ref, W_ref, b_ref, o_ref):
    x = e_ref[...]
    x = 0.5 * x * (1.0 + lax.erf(x * (2.0 ** -0.5)))
    y = lax.dot(x.astype(jnp.bfloat16), W_ref[...].astype(jnp.bfloat16),
                preferred_element_type=jnp.float32) + b_ref[...]
    o_ref[...] = 0.5 * y * (1.0 + lax.erf(y * (2.0 ** -0.5)))

  return pl.pallas_call(
      body,
      grid=(E // blk,),
      in_specs=[
          pl.BlockSpec((blk, D), lambda i: (i, 0)),
          pl.BlockSpec((D, D), lambda i: (0, 0)),
          pl.BlockSpec((1, D), lambda i: (0, 0)),
      ],
      out_specs=pl.BlockSpec((blk, D), lambda i: (i, 0)),
      out_shape=jax.ShapeDtypeStruct((E, D), jnp.float32),
  )(ein, W2, b2.reshape(1, D))


def _mean_proj(sums, counts, W3, b3, blk=512):
  G, D = sums.shape
  Dout = W3.shape[1]

  def body(s_ref, c_ref, W_ref, b_ref, o_ref):
    c = c_ref[...]
    mean = s_ref[...] / jnp.maximum(c, 1.0)
    o_ref[...] = (lax.dot(mean, W_ref[...], preferred_element_type=jnp.float32)
                  + b_ref[...] * (c > 0))

  return pl.pallas_call(
      body,
      grid=(G // blk,),
      in_specs=[
          pl.BlockSpec((blk, D), lambda i: (i, 0)),
          pl.BlockSpec((blk, 1), lambda i: (i, 0)),
          pl.BlockSpec((D, Dout), lambda i: (0, 0)),
          pl.BlockSpec((1, Dout), lambda i: (0, 0)),
      ],
      out_specs=pl.BlockSpec((blk, Dout), lambda i: (i, 0)),
      out_shape=jax.ShapeDtypeStruct((G, Dout), jnp.float32),
  )(sums, counts.reshape(G, 1), W3, b3.reshape(1, Dout))


def kernel(mesh_pos, grid_pos, mesh_to_grid_edges, W1, b1, W2, b2, W3, b3):
  grid_idx = mesh_to_grid_edges[:, 0].astype(jnp.int32)
  mesh_idx = mesh_to_grid_edges[:, 1].astype(jnp.int32)

  mp = jnp.concatenate(
      [mesh_pos, jnp.zeros((N_MESH_PAD - N_MESH, NDIM), jnp.float32)])
  h_mesh = _embed_proj(mp, W1[:DIM], jnp.zeros_like(b1))[:N_MESH]
  h_grid = _embed_proj(grid_pos, W1[DIM:], b1)

  midx_p = jnp.concatenate(
      [mesh_idx, jnp.zeros((E_PAD - N_EDGES,), jnp.int32)])
  gidx_p = jnp.concatenate(
      [grid_idx, jnp.full((E_PAD - N_EDGES,), NUM_GRID - 1, jnp.int32)])

  edge_in = _sc_gather(h_mesh, h_grid, midx_p, gidx_p)

  g2 = _edge_mlp(edge_in, W2, b2)

  cnt = jnp.zeros((NUM_GRID,), jnp.int32).at[grid_idx].add(
      1, mode="drop", unique_indices=False, indices_are_sorted=True)
  indptr = jnp.concatenate([
      jnp.zeros((1,), jnp.int32),
      jnp.cumsum(cnt, dtype=jnp.int32),
      jnp.full((7,), N_EDGES, jnp.int32),
  ])

  sums_v, counts_v = _sc_segsum(
      g2.reshape(E_PAD * 2, 128), gidx_p, indptr)
  sums = sums_v.reshape(NUM_GRID, DIM * 2)
  counts = counts_v.reshape(NUM_GRID)

  out = _mean_proj(sums, counts, W3, b3)
  return out.reshape(-1, NUM_GRID, DIM)
